# baseline (speedup 1.0000x reference)
import functools

import jax
import jax.numpy as jnp
from jax import lax
from jax.experimental import pallas as pl
from jax.experimental.pallas import tpu as pltpu
from jax.experimental.pallas import tpu_sc as plsc

_EPS = 1e-5


def _gelu(t):
    return 0.5 * t * (1.0 + lax.erf(t * 0.7071067811865476))


def _bnorm(t, g, b):
    mu = jnp.mean(t, axis=0, keepdims=True)
    var = jnp.mean((t - mu) * (t - mu), axis=0, keepdims=True)
    return (t - mu) / jnp.sqrt(var + _EPS) * g + b


def _node_prep_body(gd_ref, w1_ref, b1_ref, g1_ref, be1_ref, wmsg_ref, bmsg_ref,
                    p_ref, q_ref):
    x = jnp.dot(gd_ref[...], w1_ref[...], preferred_element_type=jnp.float32)
    x = _bnorm(x + b1_ref[...], g1_ref[...], be1_ref[...])
    x = _gelu(x)
    d = x.shape[1]
    wa = wmsg_ref[:d, :]
    wb = wmsg_ref[d:, :]
    p_ref[...] = jnp.dot(x, wa - wb, preferred_element_type=jnp.float32) + bmsg_ref[...]
    q_ref[...] = jnp.dot(x, wb, preferred_element_type=jnp.float32)


def _gate_body(ea_ref, wg_ref, bg_ref, gate_ref):
    z = jnp.dot(ea_ref[...], wg_ref[...], preferred_element_type=jnp.float32)
    gate_ref[...] = jax.nn.sigmoid(z + bg_ref[...])


def _tail_body(acc_ref, gd_ref, gbn_ref, bbn_ref, wm_ref, bm_ref, gm_ref, bem_ref,
               wl_ref, bl_ref, gl_ref, bel_ref, w2_ref, b2_ref, g2_ref, be2_ref,
               out_ref):
    npts = gd_ref.shape[0]
    agg = acc_ref[0, :npts] + acc_ref[1, :npts]
    y = _gelu(_bnorm(agg, gbn_ref[...], bbn_ref[...]))
    h = jnp.dot(y, wm_ref[...], preferred_element_type=jnp.float32) + bm_ref[...]
    h = _bnorm(h, gm_ref[...], bem_ref[...])
    att = jax.nn.sigmoid(jnp.max(h, axis=1, keepdims=True))
    y2 = jnp.dot(y * att, wl_ref[...], preferred_element_type=jnp.float32) + bl_ref[...]
    y2 = _bnorm(y2, gl_ref[...], bel_ref[...])
    out = jnp.dot(y2, w2_ref[...], preferred_element_type=jnp.float32) + b2_ref[...]
    out_ref[...] = _bnorm(out, g2_ref[...], be2_ref[...]) + gd_ref[...]


def _sc_edge_aggregate(p_nodes, q_nodes, gate, src, dst):
    n, d = p_nodes.shape
    e = src.shape[0]
    ncores, nsub = 2, 16
    nw = ncores * nsub
    chunk = 16
    edges_per_tile = e // nw
    nchunks = edges_per_tile // chunk
    assert nchunks * chunk * nw == e
    assert nchunks % 2 == 1 and nchunks >= 5
    nodes_per_tile = ((n + nsub * 8 - 1) // (nsub * 8)) * 8
    n_pad = nodes_per_tile * nsub
    zeros_blk = jnp.zeros((nodes_per_tile, d), jnp.float32)

    mesh = plsc.VectorSubcoreMesh(core_axis_name="c", subcore_axis_name="s",
                                  num_cores=ncores, num_subcores=nsub)

    @functools.partial(
        pl.kernel,
        out_type=jax.ShapeDtypeStruct((ncores, n_pad, d), jnp.float32),
        mesh=mesh,
        scratch_types=(
            [pltpu.VMEM((chunk,), jnp.int32)] * 8 +
            [pltpu.VMEM((chunk, d), jnp.float32)] * 12 +
            [pltpu.VMEM((chunk, d), jnp.float32)] * 2 +
            [pltpu.VMEM((chunk,), jnp.int32)] * 2 +
            [pltpu.VMEM_SHARED((n_pad, d), jnp.float32)] +
            [pltpu.SemaphoreType.DMA] * 18
        ),
    )
    def sc_kernel(p_hbm, q_hbm, gate_hbm, src_hbm, dst_hbm, z_hbm, out_hbm,
                  *refs):
        src_b, dst_b = refs[0:4], refs[4:8]
        p_v, q_v, g_v = refs[8:12], refs[12:16], refs[16:20]
        pr_v, dr_v = refs[20:22], refs[22:24]
        acc = refs[24]
        sem_p, sem_q = refs[25:29], refs[29:33]
        sem_g, sem_i = refs[33:37], refs[37:41]
        sem_s = refs[41:43]
        c = lax.axis_index("c")
        s = lax.axis_index("s")
        w = c * nsub + s
        pltpu.sync_copy(z_hbm, acc.at[pl.ds(s * nodes_per_tile, nodes_per_tile)])
        plsc.subcore_barrier()
        ebase = w * edges_per_tile

        def issue_idx(i, b):
            e0 = ebase + i * chunk
            pltpu.async_copy(src_hbm.at[pl.ds(e0, chunk)], src_b[b], sem_i[b])
            pltpu.async_copy(dst_hbm.at[pl.ds(e0, chunk)], dst_b[b], sem_i[b])

        def wait_idx(b):
            pltpu.make_async_copy(src_hbm.at[pl.ds(0, chunk)], src_b[b],
                                  sem_i[b]).wait()
            pltpu.make_async_copy(dst_hbm.at[pl.ds(0, chunk)], dst_b[b],
                                  sem_i[b]).wait()

        def issue(i, b):
            pltpu.async_copy(p_hbm.at[dst_b[b]], p_v[b], sem_p[b])
            pltpu.async_copy(q_hbm.at[src_b[b]], q_v[b], sem_q[b])
            pltpu.async_copy(gate_hbm.at[pl.ds(ebase + i * chunk, chunk)],
                             g_v[b], sem_g[b])

        def wait_gathers(b):
            pltpu.make_async_copy(p_hbm.at[dst_b[b]], p_v[b], sem_p[b]).wait()
            pltpu.make_async_copy(q_hbm.at[src_b[b]], q_v[b], sem_q[b]).wait()
            pltpu.make_async_copy(gate_hbm.at[pl.ds(0, chunk)], g_v[b],
                                  sem_g[b]).wait()

        def compute(b, sp):
            def row_body(r, carry):
                for k in range(d // 16):
                    sl = pl.ds(k * 16, 16)
                    pr_v[sp][r, sl] = (p_v[b][r, sl] + q_v[b][r, sl]) * g_v[b][r, sl]
                return carry

            lax.fori_loop(0, chunk, row_body, 0, unroll=4)

        def scatter(sp):
            pltpu.async_copy(pr_v[sp], acc.at[dr_v[sp]], sem_s[sp], add=True)

        def wait_scatter(sp):
            pltpu.make_async_copy(pr_v[sp], acc.at[dr_v[sp]], sem_s[sp]).wait()

        def step(i, b, sp, wait_sc, pre_gather, pre_idx):
            wait_gathers(b)
            if pre_gather:
                wait_idx((b + 3) % 4)
                issue(i + 3, (b + 3) % 4)
            if wait_sc:
                wait_scatter(sp)
            dr_v[sp][...] = dst_b[b][...]
            compute(b, sp)
            scatter(sp)
            if pre_idx:
                issue_idx(i + 4, b)

        for b in range(4):
            issue_idx(b, b)
        for b in range(3):
            wait_idx(b)
            issue(b, b)
        step(0, 0, 0, False, True, True)
        step(1, 1, 1, False, True, True)

        def quad(j, carry):
            i0 = 2 + 4 * j
            for k in range(4):
                step(i0 + k, (2 + k) % 4, k % 2, True, True, True)
            return carry

        n_quads = (nchunks - 2 - 4) // 4
        lax.fori_loop(0, n_quads, quad, 0, unroll=False)

        for i in range(2 + 4 * n_quads, nchunks):
            step(i, i % 4, i % 2, True, i + 3 < nchunks, i + 4 < nchunks)

        wait_scatter((nchunks - 2) % 2)
        wait_scatter((nchunks - 1) % 2)
        plsc.subcore_barrier()
        pltpu.sync_copy(acc.at[pl.ds(s * nodes_per_tile, nodes_per_tile)],
                        out_hbm.at[c, pl.ds(s * nodes_per_tile, nodes_per_tile)])

    return sc_kernel(p_nodes, q_nodes, gate, src, dst, zeros_blk)


def kernel(graph_data, edge_index, edge_attr, params):
    p = params
    n, d = graph_data.shape
    e = edge_index.shape[1]
    de = edge_attr.shape[1]

    def row(v):
        return v.reshape(1, -1)

    p_nodes, q_nodes = pl.pallas_call(
        _node_prep_body,
        out_shape=[jax.ShapeDtypeStruct((n, d), jnp.float32),
                   jax.ShapeDtypeStruct((n, d), jnp.float32)],
    )(graph_data, p['W1'], row(p['b1']), row(p['g1']), row(p['be1']),
      p['Wmsg'], row(p['bmsg']))

    eb = 3200
    grid = e // eb
    gate = pl.pallas_call(
        _gate_body,
        grid=(grid,),
        in_specs=[pl.BlockSpec((eb, de), lambda i: (i, 0)),
                  pl.BlockSpec((de, d), lambda i: (0, 0)),
                  pl.BlockSpec((1, d), lambda i: (0, 0))],
        out_specs=pl.BlockSpec((eb, d), lambda i: (i, 0)),
        out_shape=jax.ShapeDtypeStruct((e, d), jnp.float32),
    )(edge_attr, p['Wgate'], row(p['bgate']))

    acc = _sc_edge_aggregate(p_nodes, q_nodes, gate, edge_index[0], edge_index[1])

    out = pl.pallas_call(
        _tail_body,
        out_shape=jax.ShapeDtypeStruct((n, d), jnp.float32),
    )(acc, graph_data, row(p['gbn']), row(p['bbn']), p['Wm'], row(p['bm']),
      row(p['gm']), row(p['bem']), p['Wl'], row(p['bl']), row(p['gl']),
      row(p['bel']), p['W2'], row(p['b2']), row(p['g2']), row(p['be2']))
    return out


# speedup vs baseline: 1.4832x; 1.4832x over previous
import functools

import jax
import jax.numpy as jnp
from jax import lax
from jax.experimental import pallas as pl
from jax.experimental.pallas import tpu as pltpu
from jax.experimental.pallas import tpu_sc as plsc

_EPS = 1e-5


def _gelu(t):
    return 0.5 * t * (1.0 + lax.erf(t * 0.7071067811865476))


def _bnorm(t, g, b):
    mu = jnp.mean(t, axis=0, keepdims=True)
    var = jnp.mean((t - mu) * (t - mu), axis=0, keepdims=True)
    return (t - mu) / jnp.sqrt(var + _EPS) * g + b


def _node_prep_body(gd_ref, w1_ref, b1_ref, g1_ref, be1_ref, wmsg_ref, bmsg_ref,
                    p_ref, q_ref):
    x = jnp.dot(gd_ref[...], w1_ref[...], preferred_element_type=jnp.float32)
    x = _bnorm(x + b1_ref[...], g1_ref[...], be1_ref[...])
    x = _gelu(x)
    d = x.shape[1]
    wa = wmsg_ref[:d, :]
    wb = wmsg_ref[d:, :]
    p_ref[...] = jnp.dot(x, wa - wb, preferred_element_type=jnp.float32) + bmsg_ref[...]
    q_ref[...] = jnp.dot(x, wb, preferred_element_type=jnp.float32)


def _gate_body(ea_ref, wg_ref, bg_ref, gate_ref):
    z = jnp.dot(ea_ref[...], wg_ref[...], preferred_element_type=jnp.float32)
    gate_ref[...] = jax.nn.sigmoid(z + bg_ref[...])


def _tail_body(acc_ref, gd_ref, gbn_ref, bbn_ref, wm_ref, bm_ref, gm_ref, bem_ref,
               wl_ref, bl_ref, gl_ref, bel_ref, w2_ref, b2_ref, g2_ref, be2_ref,
               out_ref):
    npts = gd_ref.shape[0]
    agg = acc_ref[0, :npts] + acc_ref[1, :npts]
    y = _gelu(_bnorm(agg, gbn_ref[...], bbn_ref[...]))
    h = jnp.dot(y, wm_ref[...], preferred_element_type=jnp.float32) + bm_ref[...]
    h = _bnorm(h, gm_ref[...], bem_ref[...])
    att = jax.nn.sigmoid(jnp.max(h, axis=1, keepdims=True))
    y2 = jnp.dot(y * att, wl_ref[...], preferred_element_type=jnp.float32) + bl_ref[...]
    y2 = _bnorm(y2, gl_ref[...], bel_ref[...])
    out = jnp.dot(y2, w2_ref[...], preferred_element_type=jnp.float32) + b2_ref[...]
    out_ref[...] = _bnorm(out, g2_ref[...], be2_ref[...]) + gd_ref[...]


def _sc_edge_aggregate(p_nodes, q_nodes, gate, src, dst):
    n, d = p_nodes.shape
    e = src.shape[0]
    ncores, nsub = 2, 16
    nw = ncores * nsub
    chunk = 16
    edges_per_tile = e // nw
    nchunks = edges_per_tile // chunk
    assert nchunks * chunk * nw == e
    assert nchunks % 2 == 1 and nchunks >= 5
    nodes_per_tile = ((n + nsub * 8 - 1) // (nsub * 8)) * 8
    n_pad = nodes_per_tile * nsub
    zeros_blk = jnp.zeros((nodes_per_tile, d), jnp.float32)

    mesh = plsc.VectorSubcoreMesh(core_axis_name="c", subcore_axis_name="s",
                                  num_cores=ncores, num_subcores=nsub)

    @functools.partial(
        pl.kernel,
        out_type=jax.ShapeDtypeStruct((ncores, n_pad, d), jnp.float32),
        mesh=mesh,
        scratch_types=(
            [pltpu.VMEM((chunk,), jnp.int32)] * 8 +
            [pltpu.VMEM((chunk, d), jnp.float32)] * 12 +
            [pltpu.VMEM((chunk, d), jnp.float32)] * 2 +
            [pltpu.VMEM((chunk,), jnp.int32)] * 2 +
            [pltpu.VMEM_SHARED((n_pad, d), jnp.float32)] +
            [pltpu.SemaphoreType.DMA] * 18
        ),
    )
    def sc_kernel(p_hbm, q_hbm, gate_hbm, src_hbm, dst_hbm, z_hbm, out_hbm,
                  *refs):
        src_b, dst_b = refs[0:4], refs[4:8]
        p_v, q_v, g_v = refs[8:12], refs[12:16], refs[16:20]
        pr_v, dr_v = refs[20:22], refs[22:24]
        acc = refs[24]
        sem_p, sem_q = refs[25:29], refs[29:33]
        sem_g, sem_i = refs[33:37], refs[37:41]
        sem_s = refs[41:43]
        c = lax.axis_index("c")
        s = lax.axis_index("s")
        w = c * nsub + s
        pltpu.sync_copy(z_hbm, acc.at[pl.ds(s * nodes_per_tile, nodes_per_tile)])
        plsc.subcore_barrier()
        ebase = w * edges_per_tile

        def issue_idx(i, b):
            e0 = ebase + i * chunk
            pltpu.async_copy(src_hbm.at[pl.ds(e0, chunk)], src_b[b], sem_i[b])
            pltpu.async_copy(dst_hbm.at[pl.ds(e0, chunk)], dst_b[b], sem_i[b])

        def wait_idx(b):
            pltpu.make_async_copy(src_hbm.at[pl.ds(0, chunk)], src_b[b],
                                  sem_i[b]).wait()
            pltpu.make_async_copy(dst_hbm.at[pl.ds(0, chunk)], dst_b[b],
                                  sem_i[b]).wait()

        def issue(i, b):
            pltpu.async_copy(p_hbm.at[dst_b[b]], p_v[b], sem_p[b])
            pltpu.async_copy(q_hbm.at[src_b[b]], q_v[b], sem_q[b])
            pltpu.async_copy(gate_hbm.at[pl.ds(ebase + i * chunk, chunk)],
                             g_v[b], sem_g[b])

        def wait_gathers(b):
            pltpu.make_async_copy(p_hbm.at[dst_b[b]], p_v[b], sem_p[b]).wait()
            pltpu.make_async_copy(q_hbm.at[src_b[b]], q_v[b], sem_q[b]).wait()
            pltpu.make_async_copy(gate_hbm.at[pl.ds(0, chunk)], g_v[b],
                                  sem_g[b]).wait()

        def compute(b, sp):
            def row_body(r, carry):
                for k in range(d // 16):
                    sl = pl.ds(k * 16, 16)
                    pr_v[sp][r, sl] = (p_v[b][r, sl] + q_v[b][r, sl]) * g_v[b][r, sl]
                return carry

            lax.fori_loop(0, chunk, row_body, 0, unroll=False)

        def scatter(sp):
            pltpu.async_copy(pr_v[sp], acc.at[dr_v[sp]], sem_s[sp], add=True)

        def wait_scatter(sp):
            pltpu.make_async_copy(pr_v[sp], acc.at[dr_v[sp]], sem_s[sp]).wait()

        def step(i, b, sp, wait_sc, pre_gather, pre_idx):
            wait_gathers(b)
            if pre_gather:
                wait_idx((b + 3) % 4)
                issue(i + 3, (b + 3) % 4)
            if wait_sc:
                wait_scatter(sp)
            dr_v[sp][...] = dst_b[b][...]
            compute(b, sp)
            scatter(sp)
            if pre_idx:
                issue_idx(i + 4, b)

        for b in range(4):
            issue_idx(b, b)
        for b in range(3):
            wait_idx(b)
            issue(b, b)
        step(0, 0, 0, False, True, True)
        step(1, 1, 1, False, True, True)

        def quad(j, carry):
            i0 = 2 + 4 * j
            for k in range(4):
                step(i0 + k, (2 + k) % 4, k % 2, True, True, True)
            return carry

        n_quads = (nchunks - 2 - 4) // 4
        lax.fori_loop(0, n_quads, quad, 0, unroll=False)

        for i in range(2 + 4 * n_quads, nchunks):
            step(i, i % 4, i % 2, True, i + 3 < nchunks, i + 4 < nchunks)

        wait_scatter((nchunks - 2) % 2)
        wait_scatter((nchunks - 1) % 2)
        plsc.subcore_barrier()
        pltpu.sync_copy(acc.at[pl.ds(s * nodes_per_tile, nodes_per_tile)],
                        out_hbm.at[c, pl.ds(s * nodes_per_tile, nodes_per_tile)])

    return sc_kernel(p_nodes, q_nodes, gate, src, dst, zeros_blk)


def kernel(graph_data, edge_index, edge_attr, params):
    p = params
    n, d = graph_data.shape
    e = edge_index.shape[1]
    de = edge_attr.shape[1]

    def row(v):
        return v.reshape(1, -1)

    p_nodes, q_nodes = pl.pallas_call(
        _node_prep_body,
        out_shape=[jax.ShapeDtypeStruct((n, d), jnp.float32),
                   jax.ShapeDtypeStruct((n, d), jnp.float32)],
    )(graph_data, p['W1'], row(p['b1']), row(p['g1']), row(p['be1']),
      p['Wmsg'], row(p['bmsg']))

    eb = 3200
    grid = e // eb
    gate = pl.pallas_call(
        _gate_body,
        grid=(grid,),
        in_specs=[pl.BlockSpec((eb, de), lambda i: (i, 0)),
                  pl.BlockSpec((de, d), lambda i: (0, 0)),
                  pl.BlockSpec((1, d), lambda i: (0, 0))],
        out_specs=pl.BlockSpec((eb, d), lambda i: (i, 0)),
        out_shape=jax.ShapeDtypeStruct((e, d), jnp.float32),
    )(edge_attr, p['Wgate'], row(p['bgate']))

    acc = _sc_edge_aggregate(p_nodes, q_nodes, gate, edge_index[0], edge_index[1])

    out = pl.pallas_call(
        _tail_body,
        out_shape=jax.ShapeDtypeStruct((n, d), jnp.float32),
    )(acc, graph_data, row(p['gbn']), row(p['bbn']), p['Wm'], row(p['bm']),
      row(p['gm']), row(p['bem']), p['Wl'], row(p['bl']), row(p['gl']),
      row(p['bel']), p['W2'], row(p['b2']), row(p['g2']), row(p['be2']))
    return out
